# baseline (speedup 1.0000x reference)
import jax
import jax.numpy as jnp
from jax.experimental import pallas as pl


def _body(pre_ref, ctx_ref, suf_ref, out_ref):
    d = pre_ref.shape[1]
    cd = ctx_ref.shape[1]
    b = out_ref.shape[0]
    out_ref[:, 0:d] = pre_ref[...]
    out_ref[:, d:d + cd] = jnp.broadcast_to(ctx_ref[...], (b, cd))
    out_ref[:, d + cd:] = suf_ref[...]


def kernel(ctx, token_prefix, token_suffix):
    n_cls, _, d = token_prefix.shape
    n_ctx = ctx.shape[0]
    s = token_suffix.shape[1]
    seq = 1 + n_ctx + s

    pre2 = token_prefix.reshape(n_cls, d)
    suf2 = token_suffix.reshape(n_cls, s * d)
    ctx2 = ctx.reshape(1, n_ctx * d)

    B = 8
    out = pl.pallas_call(
        _body,
        grid=(n_cls // B,),
        in_specs=[
            pl.BlockSpec((B, d), lambda i: (i, 0)),
            pl.BlockSpec((1, n_ctx * d), lambda i: (0, 0)),
            pl.BlockSpec((B, s * d), lambda i: (i, 0)),
        ],
        out_specs=pl.BlockSpec((B, seq * d), lambda i: (i, 0)),
        out_shape=jax.ShapeDtypeStruct((n_cls, seq * d), jnp.float32),
    )(pre2, ctx2, suf2)
    return out.reshape(n_cls, seq, d)


# speedup vs baseline: 1.0800x; 1.0800x over previous
import jax
import jax.numpy as jnp
from jax.experimental import pallas as pl


def _body(pre_ref, ctx_ref, suf_ref, out_ref):
    d = pre_ref.shape[1]
    cd = ctx_ref.shape[1]
    b = out_ref.shape[0]
    out_ref[:, 0:d] = pre_ref[...]
    out_ref[:, d:d + cd] = jnp.broadcast_to(ctx_ref[...], (b, cd))
    out_ref[:, d + cd:] = suf_ref[...]


def kernel(ctx, token_prefix, token_suffix):
    n_cls, _, d = token_prefix.shape
    n_ctx = ctx.shape[0]
    s = token_suffix.shape[1]
    seq = 1 + n_ctx + s

    pre2 = token_prefix.reshape(n_cls, d)
    suf2 = token_suffix.reshape(n_cls, s * d)
    ctx2 = ctx.reshape(1, n_ctx * d)

    B = 40
    out = pl.pallas_call(
        _body,
        grid=(n_cls // B,),
        in_specs=[
            pl.BlockSpec((B, d), lambda i: (i, 0)),
            pl.BlockSpec((1, n_ctx * d), lambda i: (0, 0)),
            pl.BlockSpec((B, s * d), lambda i: (i, 0)),
        ],
        out_specs=pl.BlockSpec((B, seq * d), lambda i: (i, 0)),
        out_shape=jax.ShapeDtypeStruct((n_cls, seq * d), jnp.float32),
    )(pre2, ctx2, suf2)
    return out.reshape(n_cls, seq, d)
